# baseline (speedup 1.0000x reference)
import functools

import jax
import jax.numpy as jnp
from jax import lax
from jax.experimental import pallas as pl
from jax.experimental.pallas import tpu as pltpu
from jax.experimental.pallas import tpu_sc as plsc

B = 16384
H = 64
_CH = 128


@functools.lru_cache(maxsize=None)
def _sc_gather_fn(V, D, Btot):
    info = plsc.get_sparse_core_info()
    NW = info.num_cores * info.num_subcores
    b_per_w = Btot // NW
    n_ch = b_per_w // _CH
    mesh = plsc.VectorSubcoreMesh(core_axis_name="c", subcore_axis_name="s")

    @functools.partial(
        pl.kernel,
        mesh=mesh,
        out_type=jax.ShapeDtypeStruct((Btot, D), jnp.float32),
        scratch_types=[
            pltpu.VMEM((n_ch, _CH), jnp.int32),
            pltpu.VMEM((b_per_w, D), jnp.float32),
            pltpu.SemaphoreType.DMA,
        ],
        compiler_params=pltpu.CompilerParams(use_tc_tiling_on_sc=False),
    )
    def gather(table_hbm, idx_hbm, out_hbm, idx_v, rows_v, sem):
        wid = lax.axis_index("s") * info.num_cores + lax.axis_index("c")
        pltpu.sync_copy(idx_hbm.at[wid], idx_v)
        copies = []
        for j in range(n_ch):
            copies.append(
                pltpu.async_copy(
                    table_hbm.at[idx_v.at[j]],
                    rows_v.at[pl.ds(j * _CH, _CH)],
                    sem,
                )
            )
        for c in copies:
            c.wait()
        pltpu.sync_copy(rows_v, out_hbm.at[pl.ds(wid * b_per_w, b_per_w)])

    return gather


def _mlp_body(e_ref, w1_ref, b1_ref, w2_ref, b2_ref, o_ref):
    e = e_ref[...]
    h = lax.dot_general(e, w1_ref[...], (((1,), (1,)), ((), ())),
                        preferred_element_type=jnp.float32)
    h = jnp.maximum(h + b1_ref[...], 0.0)
    o = lax.dot_general(h, w2_ref[...], (((1,), (1,)), ((), ())),
                        preferred_element_type=jnp.float32)
    o_ref[...] = o + b2_ref[...]


def _tc_mlp(emb, W1, b1, W2, b2):
    Btot, D = emb.shape
    BLK = 2048
    grid = (Btot // BLK,)
    return pl.pallas_call(
        _mlp_body,
        grid=grid,
        in_specs=[
            pl.BlockSpec((BLK, D), lambda i: (i, 0)),
            pl.BlockSpec((D, D), lambda i: (0, 0)),
            pl.BlockSpec((1, D), lambda i: (0, 0)),
            pl.BlockSpec((D, D), lambda i: (0, 0)),
            pl.BlockSpec((1, D), lambda i: (0, 0)),
        ],
        out_specs=pl.BlockSpec((BLK, D), lambda i: (i, 0)),
        out_shape=jax.ShapeDtypeStruct((Btot, D), jnp.float32),
    )(emb, W1, b1.reshape(1, D), W2, b2.reshape(1, D))


def kernel(x, table, W1, b1, W2, b2):
    V, D = table.shape
    Btot = x.shape[0]
    info = plsc.get_sparse_core_info()
    NW = info.num_cores * info.num_subcores
    idx = x.reshape(NW, (Btot // NW) // _CH, _CH)
    emb = _sc_gather_fn(V, D, Btot)(table, idx)
    out = _tc_mlp(emb, W1, b1, W2, b2)
    return out.reshape(Btot, 1, D)


# speedup vs baseline: 1.6928x; 1.6928x over previous
import functools

import jax
import jax.numpy as jnp
from jax import lax
from jax.experimental import pallas as pl
from jax.experimental.pallas import tpu as pltpu
from jax.experimental.pallas import tpu_sc as plsc


@functools.lru_cache(maxsize=None)
def _sc_gather_fn(V, D, Btot):
    info = plsc.get_sparse_core_info()
    NW = info.num_cores * info.num_subcores
    b_per_w = Btot // NW
    mesh = plsc.VectorSubcoreMesh(core_axis_name="c", subcore_axis_name="s")

    @functools.partial(
        pl.kernel,
        mesh=mesh,
        out_type=jax.ShapeDtypeStruct((Btot, D), jnp.float32),
        scratch_types=[
            pltpu.VMEM((b_per_w,), jnp.int32),
            pltpu.VMEM((b_per_w, D), jnp.float32),
            pltpu.SemaphoreType.DMA,
        ],
    )
    def gather(table_hbm, idx_hbm, out_hbm, idx_v, rows_v, sem):
        wid = lax.axis_index("s") * info.num_cores + lax.axis_index("c")
        base = wid * b_per_w
        pltpu.sync_copy(idx_hbm.at[pl.ds(base, b_per_w)], idx_v)

        def issue(g, carry):
            iv = idx_v[pl.ds(g * 16, 16)]
            for j in range(16):
                pltpu.async_copy(
                    table_hbm.at[pl.ds(iv[j], 1)],
                    rows_v.at[pl.ds(g * 16 + j, 1)],
                    sem,
                )
            return carry

        lax.fori_loop(0, b_per_w // 16, issue, 0)

        def drain(i, carry):
            pltpu.make_async_copy(
                table_hbm.at[pl.ds(0, 1)], rows_v.at[pl.ds(i, 1)], sem
            ).wait()
            return carry

        lax.fori_loop(0, b_per_w, drain, 0)
        pltpu.sync_copy(rows_v, out_hbm.at[pl.ds(base, b_per_w)])

    return gather


def _mlp_body(e_ref, w1_ref, b1_ref, w2_ref, b2_ref, o_ref):
    e = e_ref[...]
    h = lax.dot_general(e, w1_ref[...], (((1,), (1,)), ((), ())),
                        preferred_element_type=jnp.float32)
    h = jnp.maximum(h + b1_ref[...], 0.0)
    o = lax.dot_general(h, w2_ref[...], (((1,), (1,)), ((), ())),
                        preferred_element_type=jnp.float32)
    o_ref[...] = o + b2_ref[...]


def _tc_mlp(emb, W1, b1, W2, b2):
    Btot, D = emb.shape
    BLK = 2048
    grid = (Btot // BLK,)
    return pl.pallas_call(
        _mlp_body,
        grid=grid,
        in_specs=[
            pl.BlockSpec((BLK, D), lambda i: (i, 0)),
            pl.BlockSpec((D, D), lambda i: (0, 0)),
            pl.BlockSpec((1, D), lambda i: (0, 0)),
            pl.BlockSpec((D, D), lambda i: (0, 0)),
            pl.BlockSpec((1, D), lambda i: (0, 0)),
        ],
        out_specs=pl.BlockSpec((BLK, D), lambda i: (i, 0)),
        out_shape=jax.ShapeDtypeStruct((Btot, D), jnp.float32),
    )(emb, W1, b1.reshape(1, D), W2, b2.reshape(1, D))


def kernel(x, table, W1, b1, W2, b2):
    V, D = table.shape
    Btot = x.shape[0]
    idx = x.reshape(Btot)
    emb = _sc_gather_fn(V, D, Btot)(table, idx)
    out = _tc_mlp(emb, W1, b1, W2, b2)
    return out.reshape(Btot, 1, D)


# speedup vs baseline: 3.4336x; 2.0284x over previous
import functools

import jax
import jax.numpy as jnp
from jax import lax
from jax.experimental import pallas as pl
from jax.experimental.pallas import tpu as pltpu
from jax.experimental.pallas import tpu_sc as plsc

_G = 6


def _iota16():
    return lax.iota(jnp.int32, 16)


def _splat(v):
    return jnp.full((16,), v, dtype=jnp.int32)


@functools.lru_cache(maxsize=None)
def _sc_gather_fn(V, D, Btot):
    info = plsc.get_sparse_core_info()
    NW = info.num_cores * info.num_subcores
    b_per_w = Btot // NW
    n_grp = b_per_w // 16
    pad = b_per_w + 32
    max_steps = -(-b_per_w // _G)
    mesh = plsc.VectorSubcoreMesh(core_axis_name="c", subcore_axis_name="s")

    @functools.partial(
        pl.kernel,
        mesh=mesh,
        out_type=jax.ShapeDtypeStruct((Btot, D), jnp.float32),
        scratch_types=[
            pltpu.VMEM((pad,), jnp.int32),
            pltpu.VMEM((pad,), jnp.int32),
            pltpu.VMEM((pad,), jnp.int32),
            pltpu.VMEM((pad,), jnp.int32),
            pltpu.VMEM((pad,), jnp.int32),
            pltpu.VMEM((pad,), jnp.int32),
            pltpu.VMEM((_G, D, 128), jnp.float32),
            pltpu.VMEM((b_per_w, D), jnp.float32),
            pltpu.SemaphoreType.DMA,
            pltpu.SemaphoreType.DMA,
        ],
        compiler_params=pltpu.CompilerParams(needs_layout_passes=False),
    )
    def gather(table_t_hbm, idxs_hbm, pos_hbm, out_hbm,
               idx_v, pos_v, tcb_v, s_v, slabs_v, starts_v,
               buf, rows_v, sem_s, sem_o):
        wid = lax.axis_index("s") * info.num_cores + lax.axis_index("c")
        base = wid * b_per_w
        pltpu.sync_copy(idxs_hbm.at[pl.ds(base, b_per_w)],
                        idx_v.at[pl.ds(0, b_per_w)])
        pltpu.sync_copy(pos_hbm.at[pl.ds(base, b_per_w)],
                        pos_v.at[pl.ds(0, b_per_w)])

        tcb_v[pl.ds(0, 16)] = _splat(-1)

        def pass_a(g, cnt):
            cur = lax.shift_right_logical(idx_v[pl.ds(g * 16, 16)], _splat(7))
            tcb_v[pl.ds(g * 16 + 1, 16)] = cur
            prev = tcb_v[pl.ds(g * 16, 16)]
            m = cur != prev
            s_rel = plsc.cumsum(m.astype(jnp.int32))
            s_v[pl.ds(g * 16, 16)] = _splat(cnt - 1) + s_rel
            plsc.store_compressed(slabs_v.at[pl.ds(cnt, 16)], cur, mask=m)
            plsc.store_compressed(starts_v.at[pl.ds(cnt, 16)],
                                  _splat(g * 16) + _iota16(), mask=m)
            return cnt + s_rel[15]

        n_slabs = lax.fori_loop(0, n_grp, pass_a, jnp.int32(0))
        starts_v[pl.ds(n_slabs, 16)] = _splat(b_per_w)

        def step(t, carry):
            s0 = t * _G

            @pl.when(s0 < n_slabs)
            def _():
                sl = slabs_v[pl.ds(s0, 16)]
                for j in range(_G):
                    @pl.when(s0 + j < n_slabs)
                    def _():
                        off = pl.multiple_of(sl[j] * 128, 128)
                        pltpu.async_copy(
                            table_t_hbm.at[:, pl.ds(off, 128)],
                            buf.at[j], sem_s,
                        )
                for j in range(_G):
                    @pl.when(s0 + j < n_slabs)
                    def _():
                        pltpu.make_async_copy(
                            table_t_hbm.at[:, pl.ds(0, 128)],
                            buf.at[j], sem_s,
                        ).wait()

                i_lo = starts_v[pl.ds(s0, 16)][0]
                hi = jnp.minimum(s0 + _G, n_slabs)
                i_hi = starts_v[pl.ds(hi, 16)][0]

                def resolve(i):
                    slot = s_v[pl.ds(i, 16)][0] - s0
                    r = idx_v[pl.ds(i, 16)][0]
                    k = lax.bitwise_and(r, 127)
                    p = pos_v[pl.ds(i, 16)][0]
                    for c0 in range(0, D, 16):
                        vals = plsc.load_gather(
                            buf,
                            [_splat(slot), c0 + _iota16(), _splat(k)],
                        )
                        rows_v[i, pl.ds(c0, 16)] = vals
                    pltpu.async_copy(
                        rows_v.at[pl.ds(i, 1)],
                        out_hbm.at[pl.ds(p, 1)],
                        sem_o,
                    )
                    return i + 1

                lax.while_loop(lambda i: i < i_hi, resolve, i_lo)

            return carry

        lax.fori_loop(0, max_steps, step, jnp.int32(0))

        def drain(i, carry):
            pltpu.make_async_copy(
                out_hbm.at[pl.ds(0, 1)], rows_v.at[pl.ds(i, 1)], sem_o
            ).wait()
            return carry

        lax.fori_loop(0, b_per_w, drain, 0)

    return gather


def _mlp_body(e_ref, w1_ref, b1_ref, w2_ref, b2_ref, o_ref):
    e = e_ref[...]
    h = lax.dot_general(e, w1_ref[...], (((1,), (1,)), ((), ())),
                        preferred_element_type=jnp.float32)
    h = jnp.maximum(h + b1_ref[...], 0.0)
    o = lax.dot_general(h, w2_ref[...], (((1,), (1,)), ((), ())),
                        preferred_element_type=jnp.float32)
    o_ref[...] = o + b2_ref[...]


def _tc_mlp(emb, W1, b1, W2, b2):
    Btot, D = emb.shape
    BLK = 2048
    grid = (Btot // BLK,)
    return pl.pallas_call(
        _mlp_body,
        grid=grid,
        in_specs=[
            pl.BlockSpec((BLK, D), lambda i: (i, 0)),
            pl.BlockSpec((D, D), lambda i: (0, 0)),
            pl.BlockSpec((1, D), lambda i: (0, 0)),
            pl.BlockSpec((D, D), lambda i: (0, 0)),
            pl.BlockSpec((1, D), lambda i: (0, 0)),
        ],
        out_specs=pl.BlockSpec((BLK, D), lambda i: (i, 0)),
        out_shape=jax.ShapeDtypeStruct((Btot, D), jnp.float32),
    )(emb, W1, b1.reshape(1, D), W2, b2.reshape(1, D))


def kernel(x, table, W1, b1, W2, b2):
    V, D = table.shape
    Btot = x.shape[0]
    idx = x.reshape(Btot)
    idx_sorted, pos = lax.sort_key_val(idx, lax.iota(jnp.int32, Btot))
    emb = _sc_gather_fn(V, D, Btot)(table.T, idx_sorted, pos)
    out = _tc_mlp(emb, W1, b1, W2, b2)
    return out.reshape(Btot, 1, D)


# speedup vs baseline: 4.0855x; 1.1898x over previous
import functools

import jax
import jax.numpy as jnp
from jax import lax
from jax.experimental import pallas as pl
from jax.experimental.pallas import tpu as pltpu
from jax.experimental.pallas import tpu_sc as plsc

_G = 6


def _iota16():
    return lax.iota(jnp.int32, 16)


def _splat(v):
    return jnp.full((16,), v, dtype=jnp.int32)


@functools.lru_cache(maxsize=None)
def _sc_gather_fn(V, D, Btot):
    info = plsc.get_sparse_core_info()
    NW = info.num_cores * info.num_subcores
    b_per_w = Btot // NW
    n_grp = b_per_w // 16
    pad = b_per_w + 32
    max_steps = -(-b_per_w // _G)
    mesh = plsc.VectorSubcoreMesh(core_axis_name="c", subcore_axis_name="s")

    @functools.partial(
        pl.kernel,
        mesh=mesh,
        out_type=jax.ShapeDtypeStruct((Btot, D), jnp.float32),
        scratch_types=[
            pltpu.VMEM((pad,), jnp.int32),
            pltpu.VMEM((pad,), jnp.int32),
            pltpu.VMEM((pad,), jnp.int32),
            pltpu.VMEM((pad,), jnp.int32),
            pltpu.VMEM((pad,), jnp.int32),
            pltpu.VMEM((pad,), jnp.int32),
            pltpu.VMEM((2, _G // 2, D, 128), jnp.float32),
            pltpu.VMEM((b_per_w, D), jnp.float32),
            pltpu.SemaphoreType.DMA,
            pltpu.SemaphoreType.DMA,
            pltpu.SemaphoreType.DMA,
        ],
        compiler_params=pltpu.CompilerParams(needs_layout_passes=False),
    )
    def gather(table_t_hbm, idxs_hbm, pos_hbm, out_hbm,
               idx_v, pos_v, tcb_v, s_v, slabs_v, starts_v,
               buf, rows_v, sem_a, sem_b, sem_o):
        wid = lax.axis_index("s") * info.num_cores + lax.axis_index("c")
        base = wid * b_per_w
        pltpu.sync_copy(idxs_hbm.at[pl.ds(base, b_per_w)],
                        idx_v.at[pl.ds(0, b_per_w)])
        pltpu.sync_copy(pos_hbm.at[pl.ds(base, b_per_w)],
                        pos_v.at[pl.ds(0, b_per_w)])

        tcb_v[pl.ds(0, 16)] = _splat(-1)

        def pass_a(g, cnt):
            cur = lax.shift_right_logical(idx_v[pl.ds(g * 16, 16)], _splat(7))
            tcb_v[pl.ds(g * 16 + 1, 16)] = cur
            prev = tcb_v[pl.ds(g * 16, 16)]
            m = cur != prev
            s_rel = plsc.cumsum(m.astype(jnp.int32))
            s_v[pl.ds(g * 16, 16)] = _splat(cnt - 1) + s_rel
            plsc.store_compressed(slabs_v.at[pl.ds(cnt, 16)], cur, mask=m)
            plsc.store_compressed(starts_v.at[pl.ds(cnt, 16)],
                                  _splat(g * 16) + _iota16(), mask=m)
            return cnt + s_rel[15]

        n_slabs = lax.fori_loop(0, n_grp, pass_a, jnp.int32(0))
        starts_v[pl.ds(n_slabs, 16)] = _splat(b_per_w)

        GH = _G // 2

        def fire_half(s0, h, sem):
            @pl.when(s0 < n_slabs)
            def _():
                sl = slabs_v[pl.ds(s0, 16)]
                for j in range(GH):
                    @pl.when(s0 + j < n_slabs)
                    def _():
                        off = pl.multiple_of(sl[j] * 128, 128)
                        pltpu.async_copy(
                            table_t_hbm.at[:, pl.ds(off, 128)],
                            buf.at[h, j], sem,
                        )

        def wait_resolve_half(s0, h, sem):
            for j in range(GH):
                @pl.when(s0 + j < n_slabs)
                def _():
                    pltpu.make_async_copy(
                        table_t_hbm.at[:, pl.ds(0, 128)],
                        buf.at[h, j], sem,
                    ).wait()

            i_lo = starts_v[pl.ds(s0, 16)][0]
            hi = jnp.minimum(s0 + GH, n_slabs)
            i_hi = starts_v[pl.ds(hi, 16)][0]

            def resolve(i):
                slot = s_v[pl.ds(i, 16)][0] - s0
                r = idx_v[pl.ds(i, 16)][0]
                k = lax.bitwise_and(r, 127)
                p = pos_v[pl.ds(i, 16)][0]
                for c0 in range(0, D, 16):
                    vals = plsc.load_gather(
                        buf,
                        [_splat(h), _splat(slot), c0 + _iota16(), _splat(k)],
                    )
                    rows_v[i, pl.ds(c0, 16)] = vals
                pltpu.async_copy(
                    rows_v.at[pl.ds(i, 1)],
                    out_hbm.at[pl.ds(p, 1)],
                    sem_o,
                )
                return i + 1

            lax.while_loop(lambda i: i < i_hi, resolve, i_lo)

        fire_half(jnp.int32(0), 0, sem_a)
        fire_half(jnp.int32(GH), 1, sem_b)

        def step(t, carry):
            s0a = t * _G

            @pl.when(s0a < n_slabs)
            def _():
                wait_resolve_half(s0a, 0, sem_a)
                fire_half(s0a + _G, 0, sem_a)
            s0b = s0a + GH

            @pl.when(s0b < n_slabs)
            def _():
                wait_resolve_half(s0b, 1, sem_b)
                fire_half(s0b + _G, 1, sem_b)

            return carry

        lax.fori_loop(0, max_steps, step, jnp.int32(0))

        def drain(i, carry):
            pltpu.make_async_copy(
                out_hbm.at[pl.ds(0, 1)], rows_v.at[pl.ds(i, 1)], sem_o
            ).wait()
            return carry

        lax.fori_loop(0, b_per_w, drain, 0)

    return gather


def _mlp_body(e_ref, w1_ref, b1_ref, w2_ref, b2_ref, o_ref):
    e = e_ref[...]
    h = lax.dot_general(e, w1_ref[...], (((1,), (1,)), ((), ())),
                        preferred_element_type=jnp.float32)
    h = jnp.maximum(h + b1_ref[...], 0.0)
    o = lax.dot_general(h, w2_ref[...], (((1,), (1,)), ((), ())),
                        preferred_element_type=jnp.float32)
    o_ref[...] = o + b2_ref[...]


def _tc_mlp(emb, W1, b1, W2, b2):
    Btot, D = emb.shape
    BLK = 2048
    grid = (Btot // BLK,)
    return pl.pallas_call(
        _mlp_body,
        grid=grid,
        in_specs=[
            pl.BlockSpec((BLK, D), lambda i: (i, 0)),
            pl.BlockSpec((D, D), lambda i: (0, 0)),
            pl.BlockSpec((1, D), lambda i: (0, 0)),
            pl.BlockSpec((D, D), lambda i: (0, 0)),
            pl.BlockSpec((1, D), lambda i: (0, 0)),
        ],
        out_specs=pl.BlockSpec((BLK, D), lambda i: (i, 0)),
        out_shape=jax.ShapeDtypeStruct((Btot, D), jnp.float32),
    )(emb, W1, b1.reshape(1, D), W2, b2.reshape(1, D))


def kernel(x, table, W1, b1, W2, b2):
    V, D = table.shape
    Btot = x.shape[0]
    idx = x.reshape(Btot)
    idx_sorted, pos = lax.sort_key_val(idx, lax.iota(jnp.int32, Btot))
    emb = _sc_gather_fn(V, D, Btot)(table.T, idx_sorted, pos)
    out = _tc_mlp(emb, W1, b1, W2, b2)
    return out.reshape(Btot, 1, D)


# speedup vs baseline: 4.2039x; 1.0290x over previous
import functools

import jax
import jax.numpy as jnp
from jax import lax
from jax.experimental import pallas as pl
from jax.experimental.pallas import tpu as pltpu
from jax.experimental.pallas import tpu_sc as plsc

_G = 6


def _iota16():
    return lax.iota(jnp.int32, 16)


def _splat(v):
    return jnp.full((16,), v, dtype=jnp.int32)


@functools.lru_cache(maxsize=None)
def _sc_gather_fn(V, D, Btot):
    info = plsc.get_sparse_core_info()
    NW = info.num_cores * info.num_subcores
    b_per_w = Btot // NW
    n_grp = b_per_w // 16
    pad = b_per_w + 32
    max_steps = -(-b_per_w // _G)
    mesh = plsc.VectorSubcoreMesh(core_axis_name="c", subcore_axis_name="s")

    @functools.partial(
        pl.kernel,
        mesh=mesh,
        out_type=jax.ShapeDtypeStruct((Btot, D), jnp.float32),
        scratch_types=[
            pltpu.VMEM((pad,), jnp.int32),
            pltpu.VMEM((pad,), jnp.int32),
            pltpu.VMEM((pad,), jnp.int32),
            pltpu.VMEM((pad,), jnp.int32),
            pltpu.VMEM((pad,), jnp.int32),
            pltpu.VMEM((pad,), jnp.int32),
            pltpu.VMEM((2, _G // 2, D, 128), jnp.float32),
            pltpu.VMEM((b_per_w, D), jnp.float32),
            pltpu.SemaphoreType.DMA,
            pltpu.SemaphoreType.DMA,
            pltpu.SemaphoreType.DMA,
        ],
        compiler_params=pltpu.CompilerParams(needs_layout_passes=False),
    )
    def gather(table_t_hbm, idxs_hbm, pos_hbm, out_hbm,
               idx_v, pos_v, tcb_v, s_v, slabs_v, starts_v,
               buf, rows_v, sem_a, sem_b, sem_o):
        wid = lax.axis_index("s") * info.num_cores + lax.axis_index("c")
        base = wid * b_per_w
        pltpu.sync_copy(idxs_hbm.at[pl.ds(base, b_per_w)],
                        idx_v.at[pl.ds(0, b_per_w)])
        pltpu.sync_copy(pos_hbm.at[pl.ds(base, b_per_w)],
                        pos_v.at[pl.ds(0, b_per_w)])

        tcb_v[pl.ds(0, 16)] = _splat(-1)

        def pass_a(g, cnt):
            cur = lax.shift_right_logical(idx_v[pl.ds(g * 16, 16)], _splat(7))
            tcb_v[pl.ds(g * 16 + 1, 16)] = cur
            prev = tcb_v[pl.ds(g * 16, 16)]
            m = cur != prev
            s_rel = plsc.cumsum(m.astype(jnp.int32))
            s_v[pl.ds(g * 16, 16)] = _splat(cnt - 1) + s_rel
            plsc.store_compressed(slabs_v.at[pl.ds(cnt, 16)], cur, mask=m)
            plsc.store_compressed(starts_v.at[pl.ds(cnt, 16)],
                                  _splat(g * 16) + _iota16(), mask=m)
            return cnt + s_rel[15]

        n_slabs = lax.fori_loop(0, n_grp, pass_a, jnp.int32(0))
        starts_v[pl.ds(n_slabs, 16)] = _splat(b_per_w)

        GH = _G // 2

        def fire_half(s0, h, sem):
            @pl.when(s0 < n_slabs)
            def _():
                sl = slabs_v[pl.ds(s0, 16)]
                for j in range(GH):
                    @pl.when(s0 + j < n_slabs)
                    def _():
                        off = pl.multiple_of(sl[j] * 128, 128)
                        pltpu.async_copy(
                            table_t_hbm.at[:, pl.ds(off, 128)],
                            buf.at[h, j], sem,
                        )

        def wait_resolve_half(s0, h, sem):
            for j in range(GH):
                @pl.when(s0 + j < n_slabs)
                def _():
                    pltpu.make_async_copy(
                        table_t_hbm.at[:, pl.ds(0, 128)],
                        buf.at[h, j], sem,
                    ).wait()

            i_lo = starts_v[pl.ds(s0, 16)][0]
            hi = jnp.minimum(s0 + GH, n_slabs)
            i_hi = starts_v[pl.ds(hi, 16)][0]

            def resolve(i):
                slot = s_v[pl.ds(i, 16)][0] - s0
                r = idx_v[pl.ds(i, 16)][0]
                k = lax.bitwise_and(r, 127)
                p = pos_v[pl.ds(i, 16)][0]
                for c0 in range(0, D, 16):
                    vals = plsc.load_gather(
                        buf,
                        [_splat(h), _splat(slot), c0 + _iota16(), _splat(k)],
                    )
                    rows_v[i, pl.ds(c0, 16)] = vals
                pltpu.async_copy(
                    rows_v.at[pl.ds(i, 1)],
                    out_hbm.at[pl.ds(p, 1)],
                    sem_o,
                )
                return i + 1

            lax.while_loop(lambda i: i < i_hi, resolve, i_lo)

        fire_half(jnp.int32(0), 0, sem_a)
        fire_half(jnp.int32(GH), 1, sem_b)

        def step(t, carry):
            s0a = t * _G

            @pl.when(s0a < n_slabs)
            def _():
                wait_resolve_half(s0a, 0, sem_a)
                fire_half(s0a + _G, 0, sem_a)
            s0b = s0a + GH

            @pl.when(s0b < n_slabs)
            def _():
                wait_resolve_half(s0b, 1, sem_b)
                fire_half(s0b + _G, 1, sem_b)

            return carry

        lax.fori_loop(0, max_steps, step, jnp.int32(0))

        def drain(i, carry):
            pltpu.make_async_copy(
                out_hbm.at[pl.ds(0, 1)], rows_v.at[pl.ds(i, 1)], sem_o
            ).wait()
            return carry

        lax.fori_loop(0, b_per_w, drain, 0)

    return gather


def _mlp_body(e_ref, w1_ref, b1_ref, w2_ref, b2_ref, o_ref):
    e = e_ref[...]
    h = lax.dot_general(e, w1_ref[...], (((1,), (1,)), ((), ())),
                        preferred_element_type=jnp.float32)
    h = jnp.maximum(h + b1_ref[...], 0.0)
    o = lax.dot_general(h, w2_ref[...], (((1,), (1,)), ((), ())),
                        preferred_element_type=jnp.float32)
    o_ref[...] = o + b2_ref[...]


def _tc_mlp(emb, W1, b1, W2, b2):
    Btot, D = emb.shape
    BLK = 8192
    grid = (Btot // BLK,)
    return pl.pallas_call(
        _mlp_body,
        grid=grid,
        in_specs=[
            pl.BlockSpec((BLK, D), lambda i: (i, 0)),
            pl.BlockSpec((D, D), lambda i: (0, 0)),
            pl.BlockSpec((1, D), lambda i: (0, 0)),
            pl.BlockSpec((D, D), lambda i: (0, 0)),
            pl.BlockSpec((1, D), lambda i: (0, 0)),
        ],
        out_specs=pl.BlockSpec((BLK, D), lambda i: (i, 0)),
        out_shape=jax.ShapeDtypeStruct((Btot, D), jnp.float32),
    )(emb, W1, b1.reshape(1, D), W2, b2.reshape(1, D))


def kernel(x, table, W1, b1, W2, b2):
    V, D = table.shape
    Btot = x.shape[0]
    idx = x.reshape(Btot)
    idx_sorted, pos = lax.sort_key_val(idx, lax.iota(jnp.int32, Btot))
    emb = _sc_gather_fn(V, D, Btot)(table.T, idx_sorted, pos)
    out = _tc_mlp(emb, W1, b1, W2, b2)
    return out.reshape(Btot, 1, D)


# speedup vs baseline: 4.3818x; 1.0423x over previous
import functools

import jax
import jax.numpy as jnp
from jax import lax
from jax.experimental import pallas as pl
from jax.experimental.pallas import tpu as pltpu
from jax.experimental.pallas import tpu_sc as plsc

_G = 6


def _iota16():
    return lax.iota(jnp.int32, 16)


def _splat(v):
    return jnp.full((16,), v, dtype=jnp.int32)


@functools.lru_cache(maxsize=None)
def _sc_gather_fn(V, D, Btot):
    info = plsc.get_sparse_core_info()
    NW = info.num_cores * info.num_subcores
    b_per_w = Btot // NW
    n_grp = b_per_w // 16
    pad = b_per_w + 32
    max_steps = -(-b_per_w // _G)
    mesh = plsc.VectorSubcoreMesh(core_axis_name="c", subcore_axis_name="s")

    @functools.partial(
        pl.kernel,
        mesh=mesh,
        out_type=jax.ShapeDtypeStruct((Btot, D), jnp.float32),
        scratch_types=[
            pltpu.VMEM((pad,), jnp.int32),
            pltpu.VMEM((pad,), jnp.int32),
            pltpu.VMEM((pad,), jnp.int32),
            pltpu.VMEM((pad,), jnp.int32),
            pltpu.VMEM((pad,), jnp.int32),
            pltpu.VMEM((pad,), jnp.int32),
            pltpu.VMEM((2, _G // 2, D, 128), jnp.float32),
            pltpu.VMEM((b_per_w, D), jnp.float32),
            pltpu.SemaphoreType.DMA,
            pltpu.SemaphoreType.DMA,
            pltpu.SemaphoreType.DMA,
        ],
        compiler_params=pltpu.CompilerParams(needs_layout_passes=False),
    )
    def gather(table_t_hbm, idxs_hbm, pos_hbm, out_hbm,
               idx_v, pos_v, tcb_v, s_v, slabs_v, starts_v,
               buf, rows_v, sem_a, sem_b, sem_o):
        wid = lax.axis_index("s") * info.num_cores + lax.axis_index("c")
        base = wid * b_per_w
        pltpu.sync_copy(idxs_hbm.at[pl.ds(base, b_per_w)],
                        idx_v.at[pl.ds(0, b_per_w)])
        pltpu.sync_copy(pos_hbm.at[pl.ds(base, b_per_w)],
                        pos_v.at[pl.ds(0, b_per_w)])

        tcb_v[pl.ds(0, 16)] = _splat(-1)

        def pass_a(g, cnt):
            cur = lax.shift_right_logical(idx_v[pl.ds(g * 16, 16)], _splat(7))
            tcb_v[pl.ds(g * 16 + 1, 16)] = cur
            prev = tcb_v[pl.ds(g * 16, 16)]
            m = cur != prev
            s_rel = plsc.cumsum(m.astype(jnp.int32))
            s_v[pl.ds(g * 16, 16)] = _splat(cnt - 1) + s_rel
            plsc.store_compressed(slabs_v.at[pl.ds(cnt, 16)], cur, mask=m)
            plsc.store_compressed(starts_v.at[pl.ds(cnt, 16)],
                                  _splat(g * 16) + _iota16(), mask=m)
            return cnt + s_rel[15]

        n_slabs = lax.fori_loop(0, n_grp, pass_a, jnp.int32(0))
        starts_v[pl.ds(n_slabs, 16)] = _splat(b_per_w)

        GH = _G // 2

        def fire_half(s0, h, sem):
            @pl.when(s0 < n_slabs)
            def _():
                sl = slabs_v[pl.ds(s0, 16)]
                for j in range(GH):
                    @pl.when(s0 + j < n_slabs)
                    def _():
                        off = pl.multiple_of(sl[j] * 128, 128)
                        pltpu.async_copy(
                            table_t_hbm.at[:, pl.ds(off, 128)],
                            buf.at[h, j], sem,
                        )

        def wait_resolve_half(s0, h, sem):
            for j in range(GH):
                @pl.when(s0 + j < n_slabs)
                def _():
                    pltpu.make_async_copy(
                        table_t_hbm.at[:, pl.ds(0, 128)],
                        buf.at[h, j], sem,
                    ).wait()

            i_lo = starts_v[pl.ds(s0, 16)][0]
            hi = jnp.minimum(s0 + GH, n_slabs)
            i_hi = starts_v[pl.ds(hi, 16)][0]

            def resolve(i):
                slot = s_v[pl.ds(i, 16)][0] - s0
                r = idx_v[pl.ds(i, 16)][0]
                k = lax.bitwise_and(r, 127)
                p = pos_v[pl.ds(i, 16)][0]
                for c0 in range(0, D, 16):
                    vals = plsc.load_gather(
                        buf,
                        [_splat(h), _splat(slot), c0 + _iota16(), _splat(k)],
                    )
                    rows_v[i, pl.ds(c0, 16)] = vals
                pltpu.async_copy(
                    rows_v.at[pl.ds(i, 1)],
                    out_hbm.at[pl.ds(p, 1)],
                    sem_o,
                )
                return i + 1

            lax.while_loop(lambda i: i < i_hi, resolve, i_lo)

        fire_half(jnp.int32(0), 0, sem_a)
        fire_half(jnp.int32(GH), 1, sem_b)

        def step(t, carry):
            s0a = t * _G

            @pl.when(s0a < n_slabs)
            def _():
                wait_resolve_half(s0a, 0, sem_a)
                fire_half(s0a + _G, 0, sem_a)
            s0b = s0a + GH

            @pl.when(s0b < n_slabs)
            def _():
                wait_resolve_half(s0b, 1, sem_b)
                fire_half(s0b + _G, 1, sem_b)

            return carry

        lax.fori_loop(0, max_steps, step, jnp.int32(0))

        def drain(i, carry):
            pltpu.make_async_copy(
                out_hbm.at[pl.ds(0, 1)], rows_v.at[pl.ds(i, 1)], sem_o
            ).wait()
            return carry

        lax.fori_loop(0, b_per_w, drain, 0)

    return gather


def _mlp_body(e_ref, w1_ref, b1_ref, w2_ref, b2_ref, o_ref):
    e = e_ref[...]
    h = lax.dot_general(e, w1_ref[...], (((1,), (1,)), ((), ())),
                        preferred_element_type=jnp.float32)
    h = jnp.maximum(h + b1_ref[...], 0.0)
    o = lax.dot_general(w2_ref[...], h, (((1,), (1,)), ((), ())),
                        preferred_element_type=jnp.float32)
    o_ref[...] = o + b2_ref[...]


def _tc_mlp(emb, W1, b1, W2, b2):
    Btot, D = emb.shape
    BLK = 8192
    grid = (Btot // BLK,)
    return pl.pallas_call(
        _mlp_body,
        grid=grid,
        in_specs=[
            pl.BlockSpec((BLK, D), lambda i: (i, 0)),
            pl.BlockSpec((D, D), lambda i: (0, 0)),
            pl.BlockSpec((1, D), lambda i: (0, 0)),
            pl.BlockSpec((D, D), lambda i: (0, 0)),
            pl.BlockSpec((D, 1), lambda i: (0, 0)),
        ],
        out_specs=pl.BlockSpec((D, BLK), lambda i: (0, i)),
        out_shape=jax.ShapeDtypeStruct((D, Btot), jnp.float32),
    )(emb, W1, b1.reshape(1, D), W2, b2.reshape(D, 1))


def kernel(x, table, W1, b1, W2, b2):
    V, D = table.shape
    Btot = x.shape[0]
    idx = x.reshape(Btot)
    idx_sorted, pos = lax.sort_key_val(idx, lax.iota(jnp.int32, Btot))
    emb = _sc_gather_fn(V, D, Btot)(table.T, idx_sorted, pos)
    out_t = _tc_mlp(emb, W1, b1, W2, b2)
    return out_t.T.reshape(Btot, 1, D)
